# baseline (speedup 1.0000x reference)
import functools

import jax
import jax.numpy as jnp
from jax import lax
from jax.experimental import pallas as pl
from jax.experimental.pallas import tpu as pltpu
from jax.experimental.pallas import tpu_sc as plsc

EMBED_D = 128
N_LAYERS = 2
N_TYPES = 3
S = 10
B = 8192

NW = 32
CH = 80
BB = 512
NCHUNK = 8
BC = B // NCHUNK


def _sc_gather(emb_0, emb_1, emb_2, sidx, idx_all):
    info = plsc.get_sparse_core_info()
    nc = info.num_cores
    seed_ch = BC // NW
    nnc = (BC * S // NW) // CH
    ng = 3 * nnc
    wsz = BC * S // NW
    mesh = plsc.VectorSubcoreMesh(core_axis_name="c", subcore_axis_name="s")

    @functools.partial(
        pl.kernel,
        mesh=mesh,
        out_type=(
            jax.ShapeDtypeStruct((BC, EMBED_D), jnp.float32),
            jax.ShapeDtypeStruct((S * BC, EMBED_D), jnp.float32),
            jax.ShapeDtypeStruct((S * BC, EMBED_D), jnp.float32),
            jax.ShapeDtypeStruct((S * BC, EMBED_D), jnp.float32),
        ),
        scratch_types=[
            pltpu.VMEM((NW, seed_ch), jnp.int32),
            pltpu.VMEM((ng, CH), jnp.int32),
            pltpu.VMEM((seed_ch, EMBED_D), jnp.float32),
            pltpu.VMEM((CH, EMBED_D), jnp.float32),
            pltpu.VMEM((CH, EMBED_D), jnp.float32),
            pltpu.VMEM((CH, EMBED_D), jnp.float32),
            pltpu.VMEM((CH, EMBED_D), jnp.float32),
            pltpu.SemaphoreType.DMA,
            pltpu.SemaphoreType.DMA,
            pltpu.SemaphoreType.DMA,
            pltpu.SemaphoreType.DMA,
            pltpu.SemaphoreType.DMA,
        ],
    )
    def k(e0, e1, e2, sidx_h, idx_h, out_s, o0, o1, o2,
          sidx_v, idx_v, srows, rows_a, rows_b, rows_c, rows_d,
          sem_s, sem_a, sem_b, sem_w1, sem_w2):
        wid = lax.axis_index("s") * nc + lax.axis_index("c")
        rows = (rows_a, rows_b, rows_c, rows_d)
        sems = (sem_a, sem_b)
        wsems = (sem_w1, sem_w2)
        tbls = (e0, e1, e2)
        outs = (o0, o1, o2)

        pltpu.sync_copy(sidx_h, sidx_v)
        seed_cp = pltpu.async_copy(e0.at[sidx_v.at[wid]], srows, sem_s)

        def dst_ref(g):
            ti, j = divmod(g, nnc)
            return outs[ti].at[pl.ds(wid * wsz + j * CH, CH)]

        pltpu.sync_copy(idx_h.at[wid], idx_v)
        cp = pltpu.async_copy(tbls[0].at[idx_v.at[0]], rows[0], sems[0])
        wrs = [None, None]
        for g in range(ng):
            if wrs[g % 2] is not None:
                wrs[g % 2].wait()
            nxt = None
            if g + 1 < ng:
                nxt = pltpu.async_copy(
                    tbls[(g + 1) // nnc].at[idx_v.at[g + 1]],
                    rows[(g + 1) % 4], sems[(g + 1) % 2])
            cp.wait()
            wrs[g % 2] = pltpu.async_copy(rows[g % 4], dst_ref(g), wsems[g % 2])
            cp = nxt
            if g == 0:
                seed_cp.wait()
                pltpu.sync_copy(srows, out_s.at[pl.ds(wid * seed_ch, seed_ch)])
        wrs[0].wait()
        wrs[1].wait()

    return k(emb_0, emb_1, emb_2, sidx, idx_all)


def _tc_body(cur_ref, x0, x1, x2, wih, whh, bih, bhh, sem, out_ref):
    xs = (x0, x1, x2)
    dn = (((1,), (1,)), ((), ()))

    xbf = [x[:].reshape(S * BB, EMBED_D).astype(jnp.bfloat16) for x in xs]
    xps = []
    for l in range(N_LAYERS):
        for t in range(N_TYPES):
            xp = lax.dot_general(xbf[t], wih[l, t].astype(jnp.bfloat16), dn,
                                 preferred_element_type=jnp.float32)
            xps.append(xp + (bih[l, t] + bhh[l, t]))

    h = [None] * (N_LAYERS * N_TYPES)
    acc = [None] * (N_LAYERS * N_TYPES)
    for s in range(S):
        for k in range(N_LAYERS * N_TYPES):
            l, t = divmod(k, N_TYPES)
            xp_s = xps[k][s * BB:(s + 1) * BB, :]
            if s == 0:
                hn = jnp.tanh(xp_s)
                acc[k] = hn
            else:
                hn = jnp.tanh(
                    xp_s
                    + lax.dot_general(h[k].astype(jnp.bfloat16),
                                      whh[l, t].astype(jnp.bfloat16), dn,
                                      preferred_element_type=jnp.float32))
                acc[k] = acc[k] + hn
            h[k] = hn
    aggs = [a * (1.0 / S) for a in acc]

    cur = cur_ref[:]
    for l in range(N_LAYERS):
        s1 = sem[l, 0]
        s2 = sem[l, 1]
        cs1 = jnp.sum(cur * s1, axis=1, keepdims=True)
        lg = [cs1 + jnp.sum(cur * s2, axis=1, keepdims=True)]
        for t in range(N_TYPES):
            lg.append(cs1 + jnp.sum(aggs[l * 3 + t] * s2, axis=1, keepdims=True))
        m = jnp.maximum(jnp.maximum(lg[0], lg[1]), jnp.maximum(lg[2], lg[3]))
        e = [jnp.exp(v - m) for v in lg]
        den = e[0] + e[1] + e[2] + e[3]
        mix = (e[0] * cur + e[1] * aggs[l * 3]
               + e[2] * aggs[l * 3 + 1] + e[3] * aggs[l * 3 + 2]) / den
        cur = jnp.where(mix > 0, mix, 0.01 * mix)
    out_ref[:] = cur


def _tc_compute(cur0, g0, g1, g2, Wih, Whh, bih, bhh, sem):
    grid = (BC // BB,)
    xspec = pl.BlockSpec((S, BB, EMBED_D), lambda i: (0, i, 0))
    wspec = pl.BlockSpec((N_LAYERS, N_TYPES, EMBED_D, EMBED_D),
                         lambda i: (0, 0, 0, 0))
    bspec = pl.BlockSpec((N_LAYERS, N_TYPES, 1, EMBED_D), lambda i: (0, 0, 0, 0))
    sspec = pl.BlockSpec((N_LAYERS, 2, 1, EMBED_D), lambda i: (0, 0, 0, 0))
    return pl.pallas_call(
        _tc_body,
        grid=grid,
        in_specs=[
            pl.BlockSpec((BB, EMBED_D), lambda i: (i, 0)),
            xspec, xspec, xspec,
            wspec, wspec, bspec, bspec, sspec,
        ],
        out_specs=pl.BlockSpec((BB, EMBED_D), lambda i: (i, 0)),
        out_shape=jax.ShapeDtypeStruct((BC, EMBED_D), jnp.float32),
    )(cur0, g0, g1, g2, Wih, Whh,
      bih.reshape(N_LAYERS, N_TYPES, 1, EMBED_D),
      bhh.reshape(N_LAYERS, N_TYPES, 1, EMBED_D),
      sem.reshape(N_LAYERS, 2, 1, EMBED_D))


def kernel(id_batch, neigh_0, neigh_1, neigh_2, emb_0, emb_1, emb_2,
           Wih, Whh, bih, bhh, sem):
    ids = id_batch.astype(jnp.int32)
    nnc = (BC * S // NW) // CH
    sidx_all = ids.reshape(NCHUNK, NW, BC // NW)
    idx_full = jnp.concatenate(
        [n.astype(jnp.int32).reshape(NCHUNK, BC, S).transpose(0, 2, 1)
         .reshape(NCHUNK, NW, nnc, CH)
         for n in (neigh_0, neigh_1, neigh_2)], axis=2)
    outs = []
    for c in range(NCHUNK):
        sidx = sidx_all[c]
        idx_all = idx_full[c]
        if c >= 2:
            sidx, _ = lax.optimization_barrier((sidx, outs[c - 2]))
        cur0, g0, g1, g2 = _sc_gather(emb_0, emb_1, emb_2, sidx, idx_all)
        outs.append(_tc_compute(cur0,
                                g0.reshape(S, BC, EMBED_D),
                                g1.reshape(S, BC, EMBED_D),
                                g2.reshape(S, BC, EMBED_D),
                                Wih, Whh, bih, bhh, sem))
    return jnp.concatenate(outs, axis=0)


# speedup vs baseline: 1.0953x; 1.0953x over previous
import functools

import jax
import jax.numpy as jnp
from jax import lax
from jax.experimental import pallas as pl
from jax.experimental.pallas import tpu as pltpu
from jax.experimental.pallas import tpu_sc as plsc

EMBED_D = 128
N_LAYERS = 2
N_TYPES = 3
S = 10
B = 8192

NW = 32
CH = 128
BB = 512
NCHUNK = 4
BC = B // NCHUNK


def _sc_gather(emb_0, emb_1, emb_2, sidx, idx_all):
    info = plsc.get_sparse_core_info()
    nc = info.num_cores
    seed_ch = BC // NW
    nnc = (BC * S // NW) // CH
    ng = 3 * nnc
    wsz = BC * S // NW
    mesh = plsc.VectorSubcoreMesh(core_axis_name="c", subcore_axis_name="s")

    @functools.partial(
        pl.kernel,
        mesh=mesh,
        out_type=(
            jax.ShapeDtypeStruct((BC, EMBED_D), jnp.float32),
            jax.ShapeDtypeStruct((S * BC, EMBED_D), jnp.float32),
            jax.ShapeDtypeStruct((S * BC, EMBED_D), jnp.float32),
            jax.ShapeDtypeStruct((S * BC, EMBED_D), jnp.float32),
        ),
        scratch_types=[
            pltpu.VMEM((NW, seed_ch), jnp.int32),
            pltpu.VMEM((ng, CH), jnp.int32),
            pltpu.VMEM((seed_ch, EMBED_D), jnp.float32),
            pltpu.VMEM((CH, EMBED_D), jnp.float32),
            pltpu.VMEM((CH, EMBED_D), jnp.float32),
            pltpu.VMEM((CH, EMBED_D), jnp.float32),
            pltpu.VMEM((CH, EMBED_D), jnp.float32),
            pltpu.SemaphoreType.DMA,
            pltpu.SemaphoreType.DMA,
            pltpu.SemaphoreType.DMA,
            pltpu.SemaphoreType.DMA,
            pltpu.SemaphoreType.DMA,
        ],
    )
    def k(e0, e1, e2, sidx_h, idx_h, out_s, o0, o1, o2,
          sidx_v, idx_v, srows, rows_a, rows_b, rows_c, rows_d,
          sem_s, sem_a, sem_b, sem_w1, sem_w2):
        wid = lax.axis_index("s") * nc + lax.axis_index("c")
        rows = (rows_a, rows_b, rows_c, rows_d)
        sems = (sem_a, sem_b)
        wsems = (sem_w1, sem_w2)
        tbls = (e0, e1, e2)
        outs = (o0, o1, o2)

        pltpu.sync_copy(sidx_h, sidx_v)
        seed_cp = pltpu.async_copy(e0.at[sidx_v.at[wid]], srows, sem_s)

        def dst_ref(g):
            ti, j = divmod(g, nnc)
            return outs[ti].at[pl.ds(wid * wsz + j * CH, CH)]

        pltpu.sync_copy(idx_h.at[wid], idx_v)
        cp = pltpu.async_copy(tbls[0].at[idx_v.at[0]], rows[0], sems[0])
        wrs = [None, None]
        for g in range(ng):
            if wrs[g % 2] is not None:
                wrs[g % 2].wait()
            nxt = None
            if g + 1 < ng:
                nxt = pltpu.async_copy(
                    tbls[(g + 1) // nnc].at[idx_v.at[g + 1]],
                    rows[(g + 1) % 4], sems[(g + 1) % 2])
            cp.wait()
            wrs[g % 2] = pltpu.async_copy(rows[g % 4], dst_ref(g), wsems[g % 2])
            cp = nxt
            if g == 0:
                seed_cp.wait()
                pltpu.sync_copy(srows, out_s.at[pl.ds(wid * seed_ch, seed_ch)])
        wrs[0].wait()
        wrs[1].wait()

    return k(emb_0, emb_1, emb_2, sidx, idx_all)


def _tc_body(cur_ref, x0, x1, x2, wih, whh, bih, bhh, sem, out_ref):
    xs = (x0, x1, x2)
    dn = (((1,), (1,)), ((), ()))

    xbf = [x[:].reshape(S * BB, EMBED_D).astype(jnp.bfloat16) for x in xs]
    xps = []
    for l in range(N_LAYERS):
        for t in range(N_TYPES):
            xp = lax.dot_general(xbf[t], wih[l, t].astype(jnp.bfloat16), dn,
                                 preferred_element_type=jnp.float32)
            xps.append(xp + (bih[l, t] + bhh[l, t]))

    h = [None] * (N_LAYERS * N_TYPES)
    acc = [None] * (N_LAYERS * N_TYPES)
    for s in range(S):
        for k in range(N_LAYERS * N_TYPES):
            l, t = divmod(k, N_TYPES)
            xp_s = xps[k][s * BB:(s + 1) * BB, :]
            if s == 0:
                hn = jnp.tanh(xp_s)
                acc[k] = hn
            else:
                hn = jnp.tanh(
                    xp_s
                    + lax.dot_general(h[k].astype(jnp.bfloat16),
                                      whh[l, t].astype(jnp.bfloat16), dn,
                                      preferred_element_type=jnp.float32))
                acc[k] = acc[k] + hn
            h[k] = hn
    aggs = [a * (1.0 / S) for a in acc]

    cur = cur_ref[:]
    for l in range(N_LAYERS):
        s1 = sem[l, 0]
        s2 = sem[l, 1]
        cs1 = jnp.sum(cur * s1, axis=1, keepdims=True)
        lg = [cs1 + jnp.sum(cur * s2, axis=1, keepdims=True)]
        for t in range(N_TYPES):
            lg.append(cs1 + jnp.sum(aggs[l * 3 + t] * s2, axis=1, keepdims=True))
        m = jnp.maximum(jnp.maximum(lg[0], lg[1]), jnp.maximum(lg[2], lg[3]))
        e = [jnp.exp(v - m) for v in lg]
        den = e[0] + e[1] + e[2] + e[3]
        mix = (e[0] * cur + e[1] * aggs[l * 3]
               + e[2] * aggs[l * 3 + 1] + e[3] * aggs[l * 3 + 2]) / den
        cur = jnp.where(mix > 0, mix, 0.01 * mix)
    out_ref[:] = cur


def _tc_compute(cur0, g0, g1, g2, Wih, Whh, bih, bhh, sem):
    grid = (BC // BB,)
    xspec = pl.BlockSpec((S, BB, EMBED_D), lambda i: (0, i, 0))
    wspec = pl.BlockSpec((N_LAYERS, N_TYPES, EMBED_D, EMBED_D),
                         lambda i: (0, 0, 0, 0))
    bspec = pl.BlockSpec((N_LAYERS, N_TYPES, 1, EMBED_D), lambda i: (0, 0, 0, 0))
    sspec = pl.BlockSpec((N_LAYERS, 2, 1, EMBED_D), lambda i: (0, 0, 0, 0))
    return pl.pallas_call(
        _tc_body,
        grid=grid,
        in_specs=[
            pl.BlockSpec((BB, EMBED_D), lambda i: (i, 0)),
            xspec, xspec, xspec,
            wspec, wspec, bspec, bspec, sspec,
        ],
        out_specs=pl.BlockSpec((BB, EMBED_D), lambda i: (i, 0)),
        out_shape=jax.ShapeDtypeStruct((BC, EMBED_D), jnp.float32),
    )(cur0, g0, g1, g2, Wih, Whh,
      bih.reshape(N_LAYERS, N_TYPES, 1, EMBED_D),
      bhh.reshape(N_LAYERS, N_TYPES, 1, EMBED_D),
      sem.reshape(N_LAYERS, 2, 1, EMBED_D))


def kernel(id_batch, neigh_0, neigh_1, neigh_2, emb_0, emb_1, emb_2,
           Wih, Whh, bih, bhh, sem):
    ids = id_batch.astype(jnp.int32)
    nnc = (BC * S // NW) // CH
    sidx_all = ids.reshape(NCHUNK, NW, BC // NW)
    idx_full = jnp.concatenate(
        [n.astype(jnp.int32).reshape(NCHUNK, BC, S).transpose(0, 2, 1)
         .reshape(NCHUNK, NW, nnc, CH)
         for n in (neigh_0, neigh_1, neigh_2)], axis=2)
    outs = []
    for c in range(NCHUNK):
        sidx = sidx_all[c]
        idx_all = idx_full[c]
        if c >= 2:
            sidx, _ = lax.optimization_barrier((sidx, outs[c - 2]))
        cur0, g0, g1, g2 = _sc_gather(emb_0, emb_1, emb_2, sidx, idx_all)
        outs.append(_tc_compute(cur0,
                                g0.reshape(S, BC, EMBED_D),
                                g1.reshape(S, BC, EMBED_D),
                                g2.reshape(S, BC, EMBED_D),
                                Wih, Whh, bih, bhh, sem))
    return jnp.concatenate(outs, axis=0)


# speedup vs baseline: 1.1051x; 1.0090x over previous
import functools

import jax
import jax.numpy as jnp
from jax import lax
from jax.experimental import pallas as pl
from jax.experimental.pallas import tpu as pltpu
from jax.experimental.pallas import tpu_sc as plsc

EMBED_D = 128
N_LAYERS = 2
N_TYPES = 3
S = 10
B = 8192

NW = 32
CH = 128
BB = 512
NCHUNK = 4
BC = B // NCHUNK


def _sc_gather(emb_0, emb_1, emb_2, sidx, idx_all):
    info = plsc.get_sparse_core_info()
    nc = info.num_cores
    seed_ch = BC // NW
    nnc = (BC * S // NW) // CH
    ng = 3 * nnc
    wsz = BC * S // NW
    mesh = plsc.VectorSubcoreMesh(core_axis_name="c", subcore_axis_name="s")

    @functools.partial(
        pl.kernel,
        mesh=mesh,
        out_type=(
            jax.ShapeDtypeStruct((BC, EMBED_D), jnp.float32),
            jax.ShapeDtypeStruct((S * BC, EMBED_D), jnp.float32),
            jax.ShapeDtypeStruct((S * BC, EMBED_D), jnp.float32),
            jax.ShapeDtypeStruct((S * BC, EMBED_D), jnp.float32),
        ),
        scratch_types=[
            pltpu.VMEM((NW, seed_ch), jnp.int32),
            pltpu.VMEM((ng, CH), jnp.int32),
            pltpu.VMEM((seed_ch, EMBED_D), jnp.float32),
            pltpu.VMEM((CH, EMBED_D), jnp.float32),
            pltpu.VMEM((CH, EMBED_D), jnp.float32),
            pltpu.VMEM((CH, EMBED_D), jnp.float32),
            pltpu.VMEM((CH, EMBED_D), jnp.float32),
            pltpu.VMEM((CH, EMBED_D), jnp.float32),
            pltpu.SemaphoreType.DMA,
            pltpu.SemaphoreType.DMA,
            pltpu.SemaphoreType.DMA,
            pltpu.SemaphoreType.DMA,
            pltpu.SemaphoreType.DMA,
            pltpu.SemaphoreType.DMA,
            pltpu.SemaphoreType.DMA,
        ],
    )
    def k(e0, e1, e2, sidx_h, idx_h, out_s, o0, o1, o2,
          sidx_v, idx_v, srows, rows_a, rows_b, rows_c, rows_d, rows_e,
          sem_s, sem_a, sem_b, sem_c, sem_w1, sem_w2, sem_w3):
        wid = lax.axis_index("s") * nc + lax.axis_index("c")
        rows = (rows_a, rows_b, rows_c, rows_d, rows_e)
        sems = (sem_a, sem_b, sem_c)
        wsems = (sem_w1, sem_w2, sem_w3)
        tbls = (e0, e1, e2)
        outs = (o0, o1, o2)

        pltpu.sync_copy(sidx_h, sidx_v)
        seed_cp = pltpu.async_copy(e0.at[sidx_v.at[wid]], srows, sem_s)

        def dst_ref(g):
            ti, j = divmod(g, nnc)
            return outs[ti].at[pl.ds(wid * wsz + j * CH, CH)]

        def gath(g):
            return pltpu.async_copy(
                tbls[g // nnc].at[idx_v.at[g]], rows[g % 5], sems[g % 3])

        pltpu.sync_copy(idx_h.at[wid], idx_v)
        gq = [gath(0), gath(1)]
        wrs = [None, None, None]
        for g in range(ng):
            if wrs[g % 3] is not None:
                wrs[g % 3].wait()
            if g + 2 < ng:
                gq.append(gath(g + 2))
            gq.pop(0).wait()
            wrs[g % 3] = pltpu.async_copy(rows[g % 5], dst_ref(g), wsems[g % 3])
            if g == 0:
                seed_cp.wait()
                pltpu.sync_copy(srows, out_s.at[pl.ds(wid * seed_ch, seed_ch)])
        for w in wrs:
            w.wait()

    return k(emb_0, emb_1, emb_2, sidx, idx_all)


def _tc_body(cur_ref, x0, x1, x2, wih, whh, bih, bhh, sem, out_ref):
    xs = (x0, x1, x2)
    dn = (((1,), (1,)), ((), ()))

    xbf = [x[:].reshape(S * BB, EMBED_D).astype(jnp.bfloat16) for x in xs]
    xps = []
    for l in range(N_LAYERS):
        for t in range(N_TYPES):
            xp = lax.dot_general(xbf[t], wih[l, t].astype(jnp.bfloat16), dn,
                                 preferred_element_type=jnp.float32)
            xps.append(xp + (bih[l, t] + bhh[l, t]))

    h = [None] * (N_LAYERS * N_TYPES)
    acc = [None] * (N_LAYERS * N_TYPES)
    for s in range(S):
        for k in range(N_LAYERS * N_TYPES):
            l, t = divmod(k, N_TYPES)
            xp_s = xps[k][s * BB:(s + 1) * BB, :]
            if s == 0:
                hn = jnp.tanh(xp_s)
                acc[k] = hn
            else:
                hn = jnp.tanh(
                    xp_s
                    + lax.dot_general(h[k].astype(jnp.bfloat16),
                                      whh[l, t].astype(jnp.bfloat16), dn,
                                      preferred_element_type=jnp.float32))
                acc[k] = acc[k] + hn
            h[k] = hn
    aggs = [a * (1.0 / S) for a in acc]

    cur = cur_ref[:]
    for l in range(N_LAYERS):
        s1 = sem[l, 0]
        s2 = sem[l, 1]
        cs1 = jnp.sum(cur * s1, axis=1, keepdims=True)
        lg = [cs1 + jnp.sum(cur * s2, axis=1, keepdims=True)]
        for t in range(N_TYPES):
            lg.append(cs1 + jnp.sum(aggs[l * 3 + t] * s2, axis=1, keepdims=True))
        m = jnp.maximum(jnp.maximum(lg[0], lg[1]), jnp.maximum(lg[2], lg[3]))
        e = [jnp.exp(v - m) for v in lg]
        den = e[0] + e[1] + e[2] + e[3]
        mix = (e[0] * cur + e[1] * aggs[l * 3]
               + e[2] * aggs[l * 3 + 1] + e[3] * aggs[l * 3 + 2]) / den
        cur = jnp.where(mix > 0, mix, 0.01 * mix)
    out_ref[:] = cur


def _tc_compute(cur0, g0, g1, g2, Wih, Whh, bih, bhh, sem):
    grid = (BC // BB,)
    xspec = pl.BlockSpec((S, BB, EMBED_D), lambda i: (0, i, 0))
    wspec = pl.BlockSpec((N_LAYERS, N_TYPES, EMBED_D, EMBED_D),
                         lambda i: (0, 0, 0, 0))
    bspec = pl.BlockSpec((N_LAYERS, N_TYPES, 1, EMBED_D), lambda i: (0, 0, 0, 0))
    sspec = pl.BlockSpec((N_LAYERS, 2, 1, EMBED_D), lambda i: (0, 0, 0, 0))
    return pl.pallas_call(
        _tc_body,
        grid=grid,
        in_specs=[
            pl.BlockSpec((BB, EMBED_D), lambda i: (i, 0)),
            xspec, xspec, xspec,
            wspec, wspec, bspec, bspec, sspec,
        ],
        out_specs=pl.BlockSpec((BB, EMBED_D), lambda i: (i, 0)),
        out_shape=jax.ShapeDtypeStruct((BC, EMBED_D), jnp.float32),
    )(cur0, g0, g1, g2, Wih, Whh,
      bih.reshape(N_LAYERS, N_TYPES, 1, EMBED_D),
      bhh.reshape(N_LAYERS, N_TYPES, 1, EMBED_D),
      sem.reshape(N_LAYERS, 2, 1, EMBED_D))


def kernel(id_batch, neigh_0, neigh_1, neigh_2, emb_0, emb_1, emb_2,
           Wih, Whh, bih, bhh, sem):
    ids = id_batch.astype(jnp.int32)
    nnc = (BC * S // NW) // CH
    sidx_all = ids.reshape(NCHUNK, NW, BC // NW)
    idx_full = jnp.concatenate(
        [n.astype(jnp.int32).reshape(NCHUNK, BC, S).transpose(0, 2, 1)
         .reshape(NCHUNK, NW, nnc, CH)
         for n in (neigh_0, neigh_1, neigh_2)], axis=2)
    outs = []
    for c in range(NCHUNK):
        sidx = sidx_all[c]
        idx_all = idx_full[c]
        if c >= 2:
            sidx, _ = lax.optimization_barrier((sidx, outs[c - 2]))
        cur0, g0, g1, g2 = _sc_gather(emb_0, emb_1, emb_2, sidx, idx_all)
        outs.append(_tc_compute(cur0,
                                g0.reshape(S, BC, EMBED_D),
                                g1.reshape(S, BC, EMBED_D),
                                g2.reshape(S, BC, EMBED_D),
                                Wih, Whh, bih, bhh, sem))
    return jnp.concatenate(outs, axis=0)


# speedup vs baseline: 1.1076x; 1.0022x over previous
import functools

import jax
import jax.numpy as jnp
from jax import lax
from jax.experimental import pallas as pl
from jax.experimental.pallas import tpu as pltpu
from jax.experimental.pallas import tpu_sc as plsc

EMBED_D = 128
N_LAYERS = 2
N_TYPES = 3
S = 10
B = 8192

NW = 32
CH = 128
BB = 512
NCHUNK = 4
BC = B // NCHUNK


def _sc_gather(emb_0, emb_1, emb_2, sidx, idx_all):
    info = plsc.get_sparse_core_info()
    nc = info.num_cores
    seed_ch = BC // NW
    nnc = (BC * S // NW) // CH
    ng = 3 * nnc
    wsz = BC * S // NW
    mesh = plsc.VectorSubcoreMesh(core_axis_name="c", subcore_axis_name="s")

    @functools.partial(
        pl.kernel,
        mesh=mesh,
        out_type=(
            jax.ShapeDtypeStruct((BC, EMBED_D), jnp.float32),
            jax.ShapeDtypeStruct((S * BC, EMBED_D), jnp.float32),
            jax.ShapeDtypeStruct((S * BC, EMBED_D), jnp.float32),
            jax.ShapeDtypeStruct((S * BC, EMBED_D), jnp.float32),
        ),
        scratch_types=[
            pltpu.VMEM((NW, seed_ch), jnp.int32),
            pltpu.VMEM((ng, CH), jnp.int32),
            pltpu.VMEM((seed_ch, EMBED_D), jnp.float32),
            pltpu.VMEM((CH, EMBED_D), jnp.float32),
            pltpu.VMEM((CH, EMBED_D), jnp.float32),
            pltpu.VMEM((CH, EMBED_D), jnp.float32),
            pltpu.VMEM((CH, EMBED_D), jnp.float32),
            pltpu.VMEM((CH, EMBED_D), jnp.float32),
            pltpu.SemaphoreType.DMA,
            pltpu.SemaphoreType.DMA,
            pltpu.SemaphoreType.DMA,
            pltpu.SemaphoreType.DMA,
            pltpu.SemaphoreType.DMA,
            pltpu.SemaphoreType.DMA,
            pltpu.SemaphoreType.DMA,
        ],
    )
    def k(e0, e1, e2, sidx_h, idx_h, out_s, o0, o1, o2,
          sidx_v, idx_v, srows, rows_a, rows_b, rows_c, rows_d, rows_e,
          sem_s, sem_a, sem_b, sem_c, sem_w1, sem_w2, sem_w3):
        wid = lax.axis_index("s") * nc + lax.axis_index("c")
        rows = (rows_a, rows_b, rows_c, rows_d, rows_e)
        sems = (sem_a, sem_b, sem_c)
        wsems = (sem_w1, sem_w2, sem_w3)
        tbls = (e0, e1, e2)
        outs = (o0, o1, o2)

        pltpu.sync_copy(sidx_h, sidx_v)
        seed_cp = pltpu.async_copy(e0.at[sidx_v.at[wid]], srows, sem_s)

        def dst_ref(g):
            ti, j = divmod(g, nnc)
            return outs[ti].at[pl.ds(wid * wsz + j * CH, CH)]

        def gath(g):
            return pltpu.async_copy(
                tbls[g // nnc].at[idx_v.at[g]], rows[g % 5], sems[g % 3])

        pltpu.sync_copy(idx_h.at[wid], idx_v)
        gq = [gath(0), gath(1)]
        wrs = [None, None, None]
        for g in range(ng):
            if wrs[g % 3] is not None:
                wrs[g % 3].wait()
            if g + 2 < ng:
                gq.append(gath(g + 2))
            gq.pop(0).wait()
            wrs[g % 3] = pltpu.async_copy(rows[g % 5], dst_ref(g), wsems[g % 3])
            if g == 0:
                seed_cp.wait()
                pltpu.sync_copy(srows, out_s.at[pl.ds(wid * seed_ch, seed_ch)])
        for w in wrs:
            w.wait()

    return k(emb_0, emb_1, emb_2, sidx, idx_all)


def _tc_body(cur_ref, x0, x1, x2, wih, whh, bih, bhh, sem, out_ref):
    xs = (x0, x1, x2)
    dn = (((1,), (1,)), ((), ()))

    xbf = [x[:].reshape(S * BB, EMBED_D).astype(jnp.bfloat16) for x in xs]
    xps = []
    for l in range(N_LAYERS):
        for t in range(N_TYPES):
            xp = lax.dot_general(xbf[t], wih[l, t].astype(jnp.bfloat16), dn,
                                 preferred_element_type=jnp.float32)
            xps.append(xp + (bih[l, t] + bhh[l, t]))

    h = [None] * (N_LAYERS * N_TYPES)
    acc = [None] * (N_LAYERS * N_TYPES)
    for l in range(N_LAYERS):
        for s in range(S):
            for t in range(N_TYPES):
                k = l * N_TYPES + t
                xp_s = xps[k][s * BB:(s + 1) * BB, :]
                if s == 0:
                    hn = jnp.tanh(xp_s)
                    acc[k] = hn
                else:
                    hn = jnp.tanh(
                        xp_s
                        + lax.dot_general(h[k].astype(jnp.bfloat16),
                                          whh[l, t].astype(jnp.bfloat16), dn,
                                          preferred_element_type=jnp.float32))
                    acc[k] = acc[k] + hn
                h[k] = hn
    aggs = [a * (1.0 / S) for a in acc]

    cur = cur_ref[:]
    for l in range(N_LAYERS):
        s1 = sem[l, 0]
        s2 = sem[l, 1]
        cs1 = jnp.sum(cur * s1, axis=1, keepdims=True)
        lg = [cs1 + jnp.sum(cur * s2, axis=1, keepdims=True)]
        for t in range(N_TYPES):
            lg.append(cs1 + jnp.sum(aggs[l * 3 + t] * s2, axis=1, keepdims=True))
        m = jnp.maximum(jnp.maximum(lg[0], lg[1]), jnp.maximum(lg[2], lg[3]))
        e = [jnp.exp(v - m) for v in lg]
        den = e[0] + e[1] + e[2] + e[3]
        mix = (e[0] * cur + e[1] * aggs[l * 3]
               + e[2] * aggs[l * 3 + 1] + e[3] * aggs[l * 3 + 2]) / den
        cur = jnp.where(mix > 0, mix, 0.01 * mix)
    out_ref[:] = cur


def _tc_compute(cur0, g0, g1, g2, Wih, Whh, bih, bhh, sem):
    grid = (BC // BB,)
    xspec = pl.BlockSpec((S, BB, EMBED_D), lambda i: (0, i, 0))
    wspec = pl.BlockSpec((N_LAYERS, N_TYPES, EMBED_D, EMBED_D),
                         lambda i: (0, 0, 0, 0))
    bspec = pl.BlockSpec((N_LAYERS, N_TYPES, 1, EMBED_D), lambda i: (0, 0, 0, 0))
    sspec = pl.BlockSpec((N_LAYERS, 2, 1, EMBED_D), lambda i: (0, 0, 0, 0))
    return pl.pallas_call(
        _tc_body,
        grid=grid,
        in_specs=[
            pl.BlockSpec((BB, EMBED_D), lambda i: (i, 0)),
            xspec, xspec, xspec,
            wspec, wspec, bspec, bspec, sspec,
        ],
        out_specs=pl.BlockSpec((BB, EMBED_D), lambda i: (i, 0)),
        out_shape=jax.ShapeDtypeStruct((BC, EMBED_D), jnp.float32),
    )(cur0, g0, g1, g2, Wih, Whh,
      bih.reshape(N_LAYERS, N_TYPES, 1, EMBED_D),
      bhh.reshape(N_LAYERS, N_TYPES, 1, EMBED_D),
      sem.reshape(N_LAYERS, 2, 1, EMBED_D))


def kernel(id_batch, neigh_0, neigh_1, neigh_2, emb_0, emb_1, emb_2,
           Wih, Whh, bih, bhh, sem):
    ids = id_batch.astype(jnp.int32)
    nnc = (BC * S // NW) // CH
    sidx_all = ids.reshape(NCHUNK, NW, BC // NW)
    idx_full = jnp.concatenate(
        [n.astype(jnp.int32).reshape(NCHUNK, BC, S).transpose(0, 2, 1)
         .reshape(NCHUNK, NW, nnc, CH)
         for n in (neigh_0, neigh_1, neigh_2)], axis=2)
    outs = []
    for c in range(NCHUNK):
        sidx = sidx_all[c]
        idx_all = idx_full[c]
        if c >= 2:
            sidx, _ = lax.optimization_barrier((sidx, outs[c - 2]))
        cur0, g0, g1, g2 = _sc_gather(emb_0, emb_1, emb_2, sidx, idx_all)
        outs.append(_tc_compute(cur0,
                                g0.reshape(S, BC, EMBED_D),
                                g1.reshape(S, BC, EMBED_D),
                                g2.reshape(S, BC, EMBED_D),
                                Wih, Whh, bih, bhh, sem))
    return jnp.concatenate(outs, axis=0)
